# Initial kernel scaffold; baseline (speedup 1.0000x reference)
import jax
import jax.numpy as jnp
from jax.experimental import pallas as pl


def kernel(x, gate_w, correction_bias, w_gate, w_up, w_down, s_gate, s_up, s_down):
    raise NotImplementedError("write your pallas kernel here")


# speedup vs baseline: 1.4242x; 1.4242x over previous
import jax
import jax.numpy as jnp
from jax.experimental import pallas as pl
from jax.experimental.pallas import tpu as pltpu

H = 1024; E = 8; F = 512; FS = 1024; N = 2048
TOP_K = 2; N_GROUP = 4; RSF = 2.5
TBLK = 256; NT = N // TBLK
ET = E + 2
LW = 128

_NEG = -1e30


def _router_body(x_ref, gw_ref, b_ref, w8_ref, idx_ref):
    xb = x_ref[...].astype(jnp.bfloat16)
    gb = gw_ref[...].astype(jnp.bfloat16)
    logits = jax.lax.dot_general(
        xb, gb, (((1,), (0,)), ((), ())),
        preferred_element_type=jnp.float32)
    lane = jax.lax.broadcasted_iota(jnp.int32, (TBLK, LW), 1)
    valid = lane < E
    scores = jax.nn.sigmoid(logits)
    sfc = scores + b_ref[...]
    sfc_m = jnp.where(valid, sfc, 0.0)
    r1 = pltpu.roll(sfc_m, LW - 1, 1)
    r2 = pltpu.roll(sfc_m, 1, 1)
    gs = sfc_m + jnp.where(lane % 2 == 0, r1, r2)
    gid = lane // 2
    grp = jnp.where(valid & (lane % 2 == 0), gs, _NEG)
    m1 = jnp.max(grp, axis=1, keepdims=True)
    g1 = jnp.min(jnp.where(grp == m1, gid, 999), axis=1, keepdims=True)
    grp2 = jnp.where(gid == g1, _NEG, grp)
    m2 = jnp.max(grp2, axis=1, keepdims=True)
    g2 = jnp.min(jnp.where(grp2 == m2, gid, 999), axis=1, keepdims=True)
    chosen = (gid == g1) | (gid == g2)
    tmp = jnp.where(chosen & valid, sfc, 0.0)
    tmp = jnp.where(valid, tmp, _NEG)
    M1 = jnp.max(tmp, axis=1, keepdims=True)
    e1 = jnp.min(jnp.where(tmp == M1, lane, 999), axis=1, keepdims=True)
    tmp2 = jnp.where(lane == e1, _NEG, tmp)
    M2 = jnp.max(tmp2, axis=1, keepdims=True)
    e2 = jnp.min(jnp.where(tmp2 == M2, lane, 999), axis=1, keepdims=True)
    w1 = jnp.sum(jnp.where(lane == e1, sfc, 0.0), axis=1, keepdims=True)
    w2 = jnp.sum(jnp.where(lane == e2, sfc, 0.0), axis=1, keepdims=True)
    den = w1 + w2 + 1e-20
    w1n = w1 / den * RSF
    w2n = w2 / den * RSF
    w8 = (jnp.where(lane == e1, w1n, 0.0) + jnp.where(lane == e2, w2n, 0.0)
          + jnp.where((lane >= E) & (lane < ET), 1.0, 0.0))
    w8_ref[...] = w8
    idx_ref[...] = (jnp.where(lane == 0, e1, 0)
                    + jnp.where(lane == 1, e2, 0)).astype(jnp.int32)


def _router(flat, gate_w, cbias):
    gwp = jnp.zeros((H, LW), jnp.float32).at[:, :E].set(gate_w)
    bp = jnp.zeros((1, LW), jnp.float32).at[0, :E].set(cbias)
    return pl.pallas_call(
        _router_body,
        grid=(NT,),
        in_specs=[
            pl.BlockSpec((TBLK, H), lambda t: (t, 0)),
            pl.BlockSpec((H, LW), lambda t: (0, 0)),
            pl.BlockSpec((1, LW), lambda t: (0, 0)),
        ],
        out_specs=[
            pl.BlockSpec((TBLK, LW), lambda t: (t, 0)),
            pl.BlockSpec((TBLK, LW), lambda t: (t, 0)),
        ],
        out_shape=[
            jax.ShapeDtypeStruct((N, LW), jnp.float32),
            jax.ShapeDtypeStruct((N, LW), jnp.int32),
        ],
    )(flat, gwp, bp)


def _ffn_body(w8_ref, xb_ref, wg_ref, wu_ref, wd_ref, o_ref):
    e = pl.program_id(0)
    t = pl.program_id(1)
    x = xb_ref[...]
    g = jnp.dot(x, wg_ref[0], preferred_element_type=jnp.float32)
    u = jnp.dot(x, wu_ref[0], preferred_element_type=jnp.float32)
    h = (g * jax.nn.sigmoid(g) * u).astype(jnp.bfloat16)
    y = jnp.dot(h, wd_ref[0], preferred_element_type=jnp.float32)
    lane = jax.lax.broadcasted_iota(jnp.int32, (TBLK, LW), 1)
    scale = jnp.sum(jnp.where(lane == e, w8_ref[...], 0.0),
                    axis=1, keepdims=True)
    y = y * scale

    @pl.when(e == 0)
    def _init():
        o_ref[pl.ds(t * TBLK, TBLK), :] = y

    @pl.when(e != 0)
    def _acc():
        o_ref[pl.ds(t * TBLK, TBLK), :] += y


def _ffn(w8, xb, wg_all, wu_all, wd_all):
    return pl.pallas_call(
        _ffn_body,
        grid=(ET, NT),
        in_specs=[
            pl.BlockSpec((TBLK, LW), lambda e, t: (t, 0)),
            pl.BlockSpec((TBLK, H), lambda e, t: (t, 0)),
            pl.BlockSpec((1, H, F), lambda e, t: (e, 0, 0)),
            pl.BlockSpec((1, H, F), lambda e, t: (e, 0, 0)),
            pl.BlockSpec((1, F, H), lambda e, t: (e, 0, 0)),
        ],
        out_specs=pl.BlockSpec((N, H), lambda e, t: (0, 0)),
        out_shape=jax.ShapeDtypeStruct((N, H), jnp.float32),
        compiler_params=pltpu.CompilerParams(
            dimension_semantics=("arbitrary", "arbitrary")),
    )(w8, xb, wg_all, wu_all, wd_all)


def kernel(x, gate_w, correction_bias, w_gate, w_up, w_down,
           s_gate, s_up, s_down):
    flat = x.reshape(N, H)
    w8, _idx = _router(flat, gate_w, correction_bias)
    sg = s_gate.reshape(H, 2, F).transpose(1, 0, 2)
    su = s_up.reshape(H, 2, F).transpose(1, 0, 2)
    sd = s_down.reshape(2, F, H)
    wg_all = jnp.concatenate([w_gate, sg], axis=0).astype(jnp.bfloat16)
    wu_all = jnp.concatenate([w_up, su], axis=0).astype(jnp.bfloat16)
    wd_all = jnp.concatenate([w_down, sd], axis=0).astype(jnp.bfloat16)
    xb = flat.astype(jnp.bfloat16)
    y = _ffn(w8, xb, wg_all, wu_all, wd_all)
    return y.reshape(1, N, H)


# speedup vs baseline: 1.6382x; 1.1503x over previous
import jax
import jax.numpy as jnp
from jax.experimental import pallas as pl
from jax.experimental.pallas import tpu as pltpu

H = 1024; E = 8; F = 512; FS = 1024; N = 2048
TOP_K = 2; N_GROUP = 4; RSF = 2.5
TBLK = 256; NT = N // TBLK
ET = E + 2
LW = 128

_NEG = -1e30


def _router_body(x_ref, gw_ref, b_ref, w8_ref, idx_ref):
    xb = x_ref[...].astype(jnp.bfloat16)
    gb = gw_ref[...].astype(jnp.bfloat16)
    logits = jax.lax.dot_general(
        xb, gb, (((1,), (0,)), ((), ())),
        preferred_element_type=jnp.float32)
    lane = jax.lax.broadcasted_iota(jnp.int32, (TBLK, LW), 1)
    valid = lane < E
    scores = jax.nn.sigmoid(logits)
    sfc = scores + b_ref[...]
    sfc_m = jnp.where(valid, sfc, 0.0)
    r1 = pltpu.roll(sfc_m, LW - 1, 1)
    r2 = pltpu.roll(sfc_m, 1, 1)
    gs = sfc_m + jnp.where(lane % 2 == 0, r1, r2)
    gid = lane // 2
    grp = jnp.where(valid & (lane % 2 == 0), gs, _NEG)
    m1 = jnp.max(grp, axis=1, keepdims=True)
    g1 = jnp.min(jnp.where(grp == m1, gid, 999), axis=1, keepdims=True)
    grp2 = jnp.where(gid == g1, _NEG, grp)
    m2 = jnp.max(grp2, axis=1, keepdims=True)
    g2 = jnp.min(jnp.where(grp2 == m2, gid, 999), axis=1, keepdims=True)
    chosen = (gid == g1) | (gid == g2)
    tmp = jnp.where(chosen & valid, sfc, 0.0)
    tmp = jnp.where(valid, tmp, _NEG)
    M1 = jnp.max(tmp, axis=1, keepdims=True)
    e1 = jnp.min(jnp.where(tmp == M1, lane, 999), axis=1, keepdims=True)
    tmp2 = jnp.where(lane == e1, _NEG, tmp)
    M2 = jnp.max(tmp2, axis=1, keepdims=True)
    e2 = jnp.min(jnp.where(tmp2 == M2, lane, 999), axis=1, keepdims=True)
    w1 = jnp.sum(jnp.where(lane == e1, sfc, 0.0), axis=1, keepdims=True)
    w2 = jnp.sum(jnp.where(lane == e2, sfc, 0.0), axis=1, keepdims=True)
    den = w1 + w2 + 1e-20
    w1n = w1 / den * RSF
    w2n = w2 / den * RSF
    w8 = (jnp.where(lane == e1, w1n, 0.0) + jnp.where(lane == e2, w2n, 0.0)
          + jnp.where((lane >= E) & (lane < ET), 1.0, 0.0))
    w8_ref[...] = w8
    idx_ref[...] = (jnp.where(lane == 0, e1, 0)
                    + jnp.where(lane == 1, e2, 0)).astype(jnp.int32)


def _router(flat, gate_w, cbias):
    gwp = jnp.zeros((H, LW), jnp.float32).at[:, :E].set(gate_w)
    bp = jnp.zeros((1, LW), jnp.float32).at[0, :E].set(cbias)
    return pl.pallas_call(
        _router_body,
        grid=(NT,),
        in_specs=[
            pl.BlockSpec((TBLK, H), lambda t: (t, 0)),
            pl.BlockSpec((H, LW), lambda t: (0, 0)),
            pl.BlockSpec((1, LW), lambda t: (0, 0)),
        ],
        out_specs=[
            pl.BlockSpec((TBLK, LW), lambda t: (t, 0)),
            pl.BlockSpec((TBLK, LW), lambda t: (t, 0)),
        ],
        out_shape=[
            jax.ShapeDtypeStruct((N, LW), jnp.float32),
            jax.ShapeDtypeStruct((N, LW), jnp.int32),
        ],
    )(flat, gwp, bp)


FBLK = 512
NFT = N // FBLK


def _ffn_body(w8_ref, xb_ref, wg_ref, wu_ref, wd_ref, o_ref):
    e = pl.program_id(0)
    t = pl.program_id(1)
    x = xb_ref[pl.ds(t * FBLK, FBLK), :]
    g = jnp.dot(x, wg_ref[0], preferred_element_type=jnp.float32)
    u = jnp.dot(x, wu_ref[0], preferred_element_type=jnp.float32)
    h = (g * jax.nn.sigmoid(g) * u).astype(jnp.bfloat16)
    y = jnp.dot(h, wd_ref[0], preferred_element_type=jnp.float32)
    lane = jax.lax.broadcasted_iota(jnp.int32, (FBLK, LW), 1)
    scale = jnp.sum(
        jnp.where(lane == e, w8_ref[pl.ds(t * FBLK, FBLK), :], 0.0),
        axis=1, keepdims=True)
    y = y * scale

    @pl.when(e == 0)
    def _init():
        o_ref[pl.ds(t * FBLK, FBLK), :] = y

    @pl.when(e != 0)
    def _acc():
        o_ref[pl.ds(t * FBLK, FBLK), :] += y


def _ffn(w8, xb, wg_all, wu_all, wd_all):
    return pl.pallas_call(
        _ffn_body,
        grid=(ET, NFT),
        in_specs=[
            pl.BlockSpec((N, LW), lambda e, t: (0, 0)),
            pl.BlockSpec((N, H), lambda e, t: (0, 0)),
            pl.BlockSpec((1, H, F), lambda e, t: (e, 0, 0)),
            pl.BlockSpec((1, H, F), lambda e, t: (e, 0, 0)),
            pl.BlockSpec((1, F, H), lambda e, t: (e, 0, 0)),
        ],
        out_specs=pl.BlockSpec((N, H), lambda e, t: (0, 0)),
        out_shape=jax.ShapeDtypeStruct((N, H), jnp.float32),
        compiler_params=pltpu.CompilerParams(
            dimension_semantics=("arbitrary", "arbitrary")),
    )(w8, xb, wg_all, wu_all, wd_all)


def kernel(x, gate_w, correction_bias, w_gate, w_up, w_down,
           s_gate, s_up, s_down):
    flat = x.reshape(N, H)
    w8, _idx = _router(flat, gate_w, correction_bias)
    sg = s_gate.reshape(H, 2, F).transpose(1, 0, 2)
    su = s_up.reshape(H, 2, F).transpose(1, 0, 2)
    sd = s_down.reshape(2, F, H)
    wg_all = jnp.concatenate([w_gate, sg], axis=0).astype(jnp.bfloat16)
    wu_all = jnp.concatenate([w_up, su], axis=0).astype(jnp.bfloat16)
    wd_all = jnp.concatenate([w_down, sd], axis=0).astype(jnp.bfloat16)
    xb = flat.astype(jnp.bfloat16)
    y = _ffn(w8, xb, wg_all, wu_all, wd_all)
    return y.reshape(1, N, H)


# speedup vs baseline: 2.1587x; 1.3177x over previous
import jax
import jax.numpy as jnp
from jax.experimental import pallas as pl
from jax.experimental.pallas import tpu as pltpu

H = 1024; E = 8; F = 512; FS = 1024; N = 2048
TOP_K = 2; N_GROUP = 4; RSF = 2.5
TBLK = 256; NT = N // TBLK
ET = E + 2
LW = 128

_NEG = -1e30


def _router_body(x_ref, gw_ref, b_ref, w8_ref, idx_ref):
    logits = jax.lax.dot_general(
        x_ref[...], gw_ref[...], (((1,), (0,)), ((), ())),
        preferred_element_type=jnp.float32)
    lane = jax.lax.broadcasted_iota(jnp.int32, (TBLK, LW), 1)
    valid = lane < E
    scores = jax.nn.sigmoid(logits)
    sfc = scores + b_ref[...]
    sfc_m = jnp.where(valid, sfc, 0.0)
    r1 = pltpu.roll(sfc_m, LW - 1, 1)
    r2 = pltpu.roll(sfc_m, 1, 1)
    gs = sfc_m + jnp.where(lane % 2 == 0, r1, r2)
    gid = lane // 2
    grp = jnp.where(valid & (lane % 2 == 0), gs, _NEG)
    m1 = jnp.max(grp, axis=1, keepdims=True)
    g1 = jnp.min(jnp.where(grp == m1, gid, 999), axis=1, keepdims=True)
    grp2 = jnp.where(gid == g1, _NEG, grp)
    m2 = jnp.max(grp2, axis=1, keepdims=True)
    g2 = jnp.min(jnp.where(grp2 == m2, gid, 999), axis=1, keepdims=True)
    chosen = (gid == g1) | (gid == g2)
    tmp = jnp.where(chosen & valid, sfc, 0.0)
    tmp = jnp.where(valid, tmp, _NEG)
    M1 = jnp.max(tmp, axis=1, keepdims=True)
    e1 = jnp.min(jnp.where(tmp == M1, lane, 999), axis=1, keepdims=True)
    tmp2 = jnp.where(lane == e1, _NEG, tmp)
    M2 = jnp.max(tmp2, axis=1, keepdims=True)
    e2 = jnp.min(jnp.where(tmp2 == M2, lane, 999), axis=1, keepdims=True)
    w1 = jnp.sum(jnp.where(lane == e1, sfc, 0.0), axis=1, keepdims=True)
    w2 = jnp.sum(jnp.where(lane == e2, sfc, 0.0), axis=1, keepdims=True)
    den = w1 + w2 + 1e-20
    w1n = w1 / den * RSF
    w2n = w2 / den * RSF
    w8 = (jnp.where(lane == e1, w1n, 0.0) + jnp.where(lane == e2, w2n, 0.0)
          + jnp.where((lane >= E) & (lane < ET), 1.0, 0.0))
    w8_ref[...] = w8
    idx_ref[...] = (jnp.where(lane == 0, e1, 0)
                    + jnp.where(lane == 1, e2, 0)).astype(jnp.int32)


def _router(flat, gate_w, cbias):
    gwp = jnp.zeros((H, LW), jnp.float32).at[:, :E].set(gate_w)
    bp = jnp.zeros((1, LW), jnp.float32).at[0, :E].set(cbias)
    return pl.pallas_call(
        _router_body,
        grid=(NT,),
        in_specs=[
            pl.BlockSpec((TBLK, H), lambda t: (t, 0)),
            pl.BlockSpec((H, LW), lambda t: (0, 0)),
            pl.BlockSpec((1, LW), lambda t: (0, 0)),
        ],
        out_specs=[
            pl.BlockSpec((TBLK, LW), lambda t: (t, 0)),
            pl.BlockSpec((TBLK, LW), lambda t: (t, 0)),
        ],
        out_shape=[
            jax.ShapeDtypeStruct((N, LW), jnp.float32),
            jax.ShapeDtypeStruct((N, LW), jnp.int32),
        ],
    )(flat, gwp, bp)


FBLK = 512
NFT = N // FBLK


def _mlp(x, wg, wu, wd):
    g = jnp.dot(x, wg, preferred_element_type=jnp.float32)
    u = jnp.dot(x, wu, preferred_element_type=jnp.float32)
    h = g * jax.nn.sigmoid(g) * u
    return jnp.dot(h, wd, preferred_element_type=jnp.float32)


def _ffn_body(w8_ref, x_ref, wg_ref, wu_ref, wd_ref,
              sg_ref, su_ref, sd_ref, o_ref):
    e = pl.program_id(0)
    t = pl.program_id(1)
    x = x_ref[pl.ds(t * FBLK, FBLK), :]
    lane = jax.lax.broadcasted_iota(jnp.int32, (FBLK, LW), 1)
    scale = jnp.sum(
        jnp.where(lane == e, w8_ref[pl.ds(t * FBLK, FBLK), :], 0.0),
        axis=1, keepdims=True)

    @pl.when(e < E)
    def _routed():
        y = _mlp(x, wg_ref[0], wu_ref[0], wd_ref[0]) * scale

        @pl.when(e == 0)
        def _init():
            o_ref[pl.ds(t * FBLK, FBLK), :] = y

        @pl.when(e != 0)
        def _acc():
            o_ref[pl.ds(t * FBLK, FBLK), :] += y

    @pl.when(e >= E)
    def _shared():
        y = _mlp(x, sg_ref[...], su_ref[...], sd_ref[...])
        o_ref[pl.ds(t * FBLK, FBLK), :] += y


def _ffn(w8, flat, w_gate, w_up, w_down, s_gate, s_up, s_down):
    cl = lambda e: jnp.minimum(e, E - 1)
    sh = lambda e: jnp.clip(e - E, 0, 1)
    return pl.pallas_call(
        _ffn_body,
        grid=(ET, NFT),
        in_specs=[
            pl.BlockSpec((N, LW), lambda e, t: (0, 0)),
            pl.BlockSpec((N, H), lambda e, t: (0, 0)),
            pl.BlockSpec((1, H, F), lambda e, t: (cl(e), 0, 0)),
            pl.BlockSpec((1, H, F), lambda e, t: (cl(e), 0, 0)),
            pl.BlockSpec((1, F, H), lambda e, t: (cl(e), 0, 0)),
            pl.BlockSpec((H, F), lambda e, t: (0, sh(e))),
            pl.BlockSpec((H, F), lambda e, t: (0, sh(e))),
            pl.BlockSpec((F, H), lambda e, t: (sh(e), 0)),
        ],
        out_specs=pl.BlockSpec((N, H), lambda e, t: (0, 0)),
        out_shape=jax.ShapeDtypeStruct((N, H), jnp.float32),
        compiler_params=pltpu.CompilerParams(
            dimension_semantics=("arbitrary", "arbitrary")),
    )(w8, flat, w_gate, w_up, w_down, s_gate, s_up, s_down)


def kernel(x, gate_w, correction_bias, w_gate, w_up, w_down,
           s_gate, s_up, s_down):
    flat = x.reshape(N, H)
    w8, _idx = _router(flat, gate_w, correction_bias)
    y = _ffn(w8, flat, w_gate, w_up, w_down, s_gate, s_up, s_down)
    return y.reshape(1, N, H)


# speedup vs baseline: 2.2991x; 1.0650x over previous
import jax
import jax.numpy as jnp
from jax.experimental import pallas as pl
from jax.experimental.pallas import tpu as pltpu

H = 1024; E = 8; F = 512; FS = 1024; N = 2048
RSF = 2.5
ET = E + 2
LW = 128
FBLK = 512
NFT = N // FBLK

_NEG = -1e30


def _route_block(x, gw, bias):
    logits = jax.lax.dot_general(
        x, gw, (((1,), (0,)), ((), ())), preferred_element_type=jnp.float32)
    lane = jax.lax.broadcasted_iota(jnp.int32, (FBLK, LW), 1)
    valid = lane < E
    scores = jax.nn.sigmoid(logits)
    sfc = scores + bias
    sfc_m = jnp.where(valid, sfc, 0.0)
    r1 = pltpu.roll(sfc_m, LW - 1, 1)
    r2 = pltpu.roll(sfc_m, 1, 1)
    gs = sfc_m + jnp.where(lane % 2 == 0, r1, r2)
    gid = lane // 2
    grp = jnp.where(valid & (lane % 2 == 0), gs, _NEG)
    m1 = jnp.max(grp, axis=1, keepdims=True)
    g1 = jnp.min(jnp.where(grp == m1, gid, 999), axis=1, keepdims=True)
    grp2 = jnp.where(gid == g1, _NEG, grp)
    m2 = jnp.max(grp2, axis=1, keepdims=True)
    g2 = jnp.min(jnp.where(grp2 == m2, gid, 999), axis=1, keepdims=True)
    chosen = (gid == g1) | (gid == g2)
    tmp = jnp.where(chosen & valid, sfc, 0.0)
    tmp = jnp.where(valid, tmp, _NEG)
    M1 = jnp.max(tmp, axis=1, keepdims=True)
    e1 = jnp.min(jnp.where(tmp == M1, lane, 999), axis=1, keepdims=True)
    tmp2 = jnp.where(lane == e1, _NEG, tmp)
    M2 = jnp.max(tmp2, axis=1, keepdims=True)
    e2 = jnp.min(jnp.where(tmp2 == M2, lane, 999), axis=1, keepdims=True)
    w1 = jnp.sum(jnp.where(lane == e1, sfc, 0.0), axis=1, keepdims=True)
    w2 = jnp.sum(jnp.where(lane == e2, sfc, 0.0), axis=1, keepdims=True)
    den = w1 + w2 + 1e-20
    return (jnp.where(lane == e1, w1 / den * RSF, 0.0)
            + jnp.where(lane == e2, w2 / den * RSF, 0.0)
            + jnp.where((lane >= E) & (lane < ET), 1.0, 0.0))


def _mlp(x, wg, wu, wd):
    g = jnp.dot(x, wg, preferred_element_type=jnp.float32)
    u = jnp.dot(x, wu, preferred_element_type=jnp.float32)
    h = g * jax.nn.sigmoid(g) * u
    return jnp.dot(h, wd, preferred_element_type=jnp.float32)


def _body(x_ref, gw_ref, b_ref,
          wgA_ref, wuA_ref, wdA_ref,
          wgB_ref, wuB_ref, wdB_ref,
          sg_ref, su_ref, sd_ref,
          o_ref, w8_ref):
    e = pl.program_id(0)
    t = pl.program_id(1)
    rows = pl.ds(t * FBLK, FBLK)
    x = x_ref[rows, :]

    @pl.when(e == 0)
    def _route():
        w8_ref[rows, :] = _route_block(x, gw_ref[...], b_ref[...])

    lane = jax.lax.broadcasted_iota(jnp.int32, (FBLK, LW), 1)
    scale = jnp.sum(jnp.where(lane == e, w8_ref[rows, :], 0.0),
                    axis=1, keepdims=True)

    @pl.when(e == 0)
    def _init():
        o_ref[rows, :] = _mlp(x, wgA_ref[0], wuA_ref[0], wdA_ref[0]) * scale

    @pl.when((e != 0) & (e < E) & (e % 2 == 0))
    def _even():
        o_ref[rows, :] += _mlp(x, wgA_ref[0], wuA_ref[0], wdA_ref[0]) * scale

    @pl.when((e < E) & (e % 2 == 1))
    def _odd():
        o_ref[rows, :] += _mlp(x, wgB_ref[0], wuB_ref[0], wdB_ref[0]) * scale

    @pl.when(e >= E)
    def _shared():
        o_ref[rows, :] += _mlp(x, sg_ref[...], su_ref[...], sd_ref[...])


def kernel(x, gate_w, correction_bias, w_gate, w_up, w_down,
           s_gate, s_up, s_down):
    flat = x.reshape(N, H)
    gwp = jnp.zeros((H, LW), jnp.float32).at[:, :E].set(gate_w)
    bp = jnp.zeros((1, LW), jnp.float32).at[0, :E].set(correction_bias)
    eA = lambda e: jnp.minimum(2 * ((e + 1) // 2), E - 2)
    eB = lambda e: jnp.minimum(2 * (e // 2) + 1, E - 1)
    sh = lambda e: jnp.clip(e - E, 0, 1)
    y = pl.pallas_call(
        _body,
        grid=(ET, NFT),
        in_specs=[
            pl.BlockSpec((N, H), lambda e, t: (0, 0)),
            pl.BlockSpec((H, LW), lambda e, t: (0, 0)),
            pl.BlockSpec((1, LW), lambda e, t: (0, 0)),
            pl.BlockSpec((1, H, F), lambda e, t: (eA(e), 0, 0)),
            pl.BlockSpec((1, H, F), lambda e, t: (eA(e), 0, 0)),
            pl.BlockSpec((1, F, H), lambda e, t: (eA(e), 0, 0)),
            pl.BlockSpec((1, H, F), lambda e, t: (eB(e), 0, 0)),
            pl.BlockSpec((1, H, F), lambda e, t: (eB(e), 0, 0)),
            pl.BlockSpec((1, F, H), lambda e, t: (eB(e), 0, 0)),
            pl.BlockSpec((H, F), lambda e, t: (0, sh(e))),
            pl.BlockSpec((H, F), lambda e, t: (0, sh(e))),
            pl.BlockSpec((F, H), lambda e, t: (sh(e), 0)),
        ],
        out_specs=pl.BlockSpec((N, H), lambda e, t: (0, 0)),
        out_shape=jax.ShapeDtypeStruct((N, H), jnp.float32),
        scratch_shapes=[pltpu.VMEM((N, LW), jnp.float32)],
        compiler_params=pltpu.CompilerParams(
            dimension_semantics=("arbitrary", "arbitrary")),
    )(flat, gwp, bp,
      w_gate, w_up, w_down,
      w_gate, w_up, w_down,
      s_gate, s_up, s_down)
    return y.reshape(1, N, H)
